# baseline (speedup 1.0000x reference)
import functools

import jax
import jax.numpy as jnp
import numpy as np
from jax import lax
from jax.experimental import pallas as pl
from jax.experimental.pallas import tpu as pltpu
from jax.experimental.pallas import tpu_sc as plsc

_A = 6
_S = 20
_OPP = 2


def _rotl(x, r):
    return (x << np.uint32(r)) | (x >> np.uint32(32 - r))


def _threefry2x32(k1, k2, x0, x1):
    ks0 = np.uint32(k1)
    ks1 = np.uint32(k2)
    ks2 = np.uint32(ks0 ^ ks1 ^ np.uint32(0x1BD11BDA))
    ks = [ks0, ks1, ks2]
    rotations = [(13, 15, 26, 6), (17, 29, 16, 24)]
    x0 = x0 + ks0
    x1 = x1 + ks1
    for i in range(5):
        for r in rotations[i % 2]:
            x0 = x0 + x1
            x1 = _rotl(x1, r)
            x1 = x1 ^ x0
        x0 = x0 + ks[(i + 1) % 3]
        x1 = x1 + ks[(i + 2) % 3] + np.uint32(i + 1)
    return x0, x1


def _fold_in(key, data):
    o0, o1 = _threefry2x32(key[0], key[1],
                           np.atleast_1d(np.uint32(0)),
                           np.atleast_1d(np.uint32(data)))
    return (o0[0], o1[0])


def _gumbel_np(key, n):
    cnt = np.arange(n, dtype=np.uint64)
    hi = (cnt >> np.uint64(32)).astype(np.uint32)
    lo = (cnt & np.uint64(0xFFFFFFFF)).astype(np.uint32)
    o0, o1 = _threefry2x32(key[0], key[1], hi, lo)
    bits = o0 ^ o1
    f = ((bits >> np.uint32(9)) | np.uint32(0x3F800000)).view(np.float32)
    u = f - np.float32(1.0)
    tiny = np.float32(np.finfo(np.float32).tiny)
    u = np.maximum(tiny, u * (np.float32(1.0) - tiny) + tiny)
    with np.errstate(divide="ignore"):
        return -np.log(-np.log(u))


@functools.lru_cache(maxsize=2)
def _gumbel_host_sc(B):
    root = (np.uint32(0), np.uint32(42))
    out = np.empty((_OPP * _S * _A, B), np.float32)
    for j in range(_OPP):
        kj = _fold_in(root, j)
        for i in range(_S):
            ki = _fold_in(kj, i)
            g = _gumbel_np(ki, B * _A).reshape(B, _A)
            out[(j * _S + i) * _A:(j * _S + i + 1) * _A] = g.T
    return out


def _zz_body(xb_ref, wcat_ref, bcat_ref, out_ref):
    zz = lax.dot_general(wcat_ref[...], xb_ref[...], (((1,), (1,)), ((), ())),
                         preferred_element_type=jnp.float32)
    out_ref[...] = zz + bcat_ref[...]


def _zz_call(x, wcat, bcat):
    B, D = x.shape
    Bb = 2048
    return pl.pallas_call(
        _zz_body,
        grid=(B // Bb,),
        in_specs=[
            pl.BlockSpec((Bb, D), lambda i: (i, 0)),
            pl.BlockSpec((24, D), lambda i: (0, 0)),
            pl.BlockSpec((24, 1), lambda i: (0, 0)),
        ],
        out_specs=pl.BlockSpec((24, Bb), lambda i: (0, i)),
        out_shape=jax.ShapeDtypeStruct((24, B), jnp.float32),
        compiler_params=pltpu.CompilerParams(
            dimension_semantics=("parallel",),
        ),
    )(x, wcat, bcat)


_NC = 2
_NS = 16
_NW = _NC * _NS
_CH = 128


def _sc_body(zz_hbm, g_hbm, ptab_hbm, out_hbm, zz_v, g_v, ptab_v, out_v):
    A, S = _A, _S
    wid = lax.axis_index("s") * _NC + lax.axis_index("c")
    B = zz_hbm.shape[1]
    rpw = B // _NW
    base = wid * rpw

    pltpu.sync_copy(ptab_hbm, ptab_v)
    pltpu.sync_copy(zz_hbm.at[:, pl.ds(base, rpw)], zz_v)

    lane = lax.iota(jnp.int32, 16)

    for c in range(rpw // _CH):
        coff = c * _CH
        pltpu.sync_copy(g_hbm.at[:, pl.ds(base + coff, _CH)], g_v)

        def grp(gi, carry):
            goff = coff + gi * 16
            gg = gi * 16
            z0 = [zz_v[a, pl.ds(goff, 16)] for a in range(A)]
            z1 = [zz_v[8 + a, pl.ds(goff, 16)] for a in range(A)]
            bs = [zz_v[16 + o, pl.ds(goff, 16)] for o in range(A)]
            m0 = z0[0]
            m1 = z1[0]
            for a in range(1, A):
                m0 = jnp.maximum(m0, z0[a])
                m1 = jnp.maximum(m1, z1[a])
            mm = m0 + m1
            acc = [jnp.zeros((16,), jnp.float32) for _ in range(A)]
            wsum = jnp.zeros((16,), jnp.float32)

            def pick(zrows, irow):
                v = [zrows[a] + g_v[irow * A + a, pl.ds(gg, 16)]
                     for a in range(A)]
                m = v[0]
                for a in range(1, A):
                    m = jnp.maximum(m, v[a])
                aidx = jnp.full((16,), A - 1, jnp.int32)
                u = zrows[A - 1]
                for a in range(A - 2, -1, -1):
                    hit = v[a] == m
                    aidx = jnp.where(hit, jnp.int32(a), aidx)
                    u = jnp.where(hit, zrows[a], u)
                return aidx, u

            for i in range(S):
                a0, u0 = pick(z0, i)
                a1, u1 = pick(z1, S + i)
                w = jnp.exp(u0 + u1 - mm)
                pidx = a0 * A + a1
                e = []
                s = None
                for o in range(A):
                    po = plsc.load_gather(ptab_v, [pidx + (o * A * A)])
                    eo = jnp.exp(bs[o] + po)
                    e.append(eo)
                    s = eo if s is None else s + eo
                r = w / s
                for o in range(A):
                    acc[o] = acc[o] + r * e[o]
                wsum = wsum + w

            orow = (goff + lane) * A
            for o in range(A):
                plsc.store_scatter(out_v, [orow + o], acc[o] / wsum)
            return carry

        lax.fori_loop(0, _CH // 16, grp, 0)

    pltpu.sync_copy(out_v, out_hbm.at[pl.ds(base * A, rpw * A)])


def _sc_call(zz, g, ptab):
    B = zz.shape[1]
    rpw = B // _NW
    mesh = plsc.VectorSubcoreMesh(core_axis_name="c", subcore_axis_name="s")
    k = functools.partial(
        pl.kernel,
        out_type=jax.ShapeDtypeStruct((B * _A,), jnp.float32),
        mesh=mesh,
        scratch_types=[
            pltpu.VMEM((24, rpw), jnp.float32),
            pltpu.VMEM((_OPP * _S * _A, _CH), jnp.float32),
            pltpu.VMEM((_A * _A * _A,), jnp.float32),
            pltpu.VMEM((rpw * _A,), jnp.float32),
        ],
        compiler_params=pltpu.CompilerParams(needs_layout_passes=False),
    )(_sc_body)
    return k(zz, g, ptab)


def kernel(x, W_opp0, b_opp0, W_opp1, b_opp1, W, b):
    B, D = x.shape
    A = _A

    g = jnp.asarray(_gumbel_host_sc(B))

    zpadW = jnp.zeros((2, D), x.dtype)
    wcat = jnp.concatenate(
        [W_opp0, zpadW, W_opp1, zpadW, W[:, :D], zpadW], axis=0)
    zpadb = jnp.zeros((2,), x.dtype)
    bcat = jnp.concatenate(
        [b_opp0, zpadb, b_opp1, zpadb, b, zpadb], axis=0)[:, None]
    c0 = W[:, D:D + A]
    c1 = W[:, D + A:D + 2 * A]
    ptab = (c0[:, :, None] + c1[:, None, :]).reshape(A * A * A)

    zz = _zz_call(x, wcat, bcat)
    outflat = _sc_call(zz, g, ptab)
    return outflat.reshape(B, A)


# speedup vs baseline: 1.3708x; 1.3708x over previous
import functools

import jax
import jax.numpy as jnp
import numpy as np
from jax import lax
from jax.experimental import pallas as pl
from jax.experimental.pallas import tpu as pltpu
from jax.experimental.pallas import tpu_sc as plsc

_A = 6
_S = 20
_OPP = 2


def _rotl(x, r):
    return (x << np.uint32(r)) | (x >> np.uint32(32 - r))


def _threefry2x32(k1, k2, x0, x1):
    ks0 = np.uint32(k1)
    ks1 = np.uint32(k2)
    ks2 = np.uint32(ks0 ^ ks1 ^ np.uint32(0x1BD11BDA))
    ks = [ks0, ks1, ks2]
    rotations = [(13, 15, 26, 6), (17, 29, 16, 24)]
    x0 = x0 + ks0
    x1 = x1 + ks1
    for i in range(5):
        for r in rotations[i % 2]:
            x0 = x0 + x1
            x1 = _rotl(x1, r)
            x1 = x1 ^ x0
        x0 = x0 + ks[(i + 1) % 3]
        x1 = x1 + ks[(i + 2) % 3] + np.uint32(i + 1)
    return x0, x1


def _fold_in(key, data):
    o0, o1 = _threefry2x32(key[0], key[1],
                           np.atleast_1d(np.uint32(0)),
                           np.atleast_1d(np.uint32(data)))
    return (o0[0], o1[0])


def _gumbel_np(key, n):
    cnt = np.arange(n, dtype=np.uint64)
    hi = (cnt >> np.uint64(32)).astype(np.uint32)
    lo = (cnt & np.uint64(0xFFFFFFFF)).astype(np.uint32)
    o0, o1 = _threefry2x32(key[0], key[1], hi, lo)
    bits = o0 ^ o1
    f = ((bits >> np.uint32(9)) | np.uint32(0x3F800000)).view(np.float32)
    u = f - np.float32(1.0)
    tiny = np.float32(np.finfo(np.float32).tiny)
    u = np.maximum(tiny, u * (np.float32(1.0) - tiny) + tiny)
    with np.errstate(divide="ignore"):
        return -np.log(-np.log(u))


@functools.lru_cache(maxsize=2)
def _gumbel_full(B):
    root = (np.uint32(0), np.uint32(42))
    out = np.empty((_OPP * _S, B, _A), np.float32)
    for j in range(_OPP):
        kj = _fold_in(root, j)
        for i in range(_S):
            ki = _fold_in(kj, i)
            out[j * _S + i] = _gumbel_np(ki, B * _A).reshape(B, _A)
    return out


@functools.lru_cache(maxsize=2)
def _gumbel_host_sc(B, lo, hi):
    g = _gumbel_full(B)[:, lo:hi, :]
    return np.ascontiguousarray(
        g.transpose(0, 2, 1).reshape(_OPP * _S * _A, hi - lo))


@functools.lru_cache(maxsize=2)
def _gumbel_host_tc(B, lo, hi, Bb):
    g = _gumbel_full(B)[:, lo:hi, :]
    nb = (hi - lo) // Bb
    g = g.reshape(_OPP * _S, nb, Bb, _A).transpose(1, 0, 3, 2)
    return np.ascontiguousarray(g)


def _tc_body(xb_ref, wcat_ref, bcat_ref, c0_ref, c1_ref, g_ref, out_ref):
    A, S = _A, _S
    xb = xb_ref[...]
    zz = lax.dot_general(wcat_ref[...], xb, (((1,), (1,)), ((), ())),
                         preferred_element_type=jnp.float32)
    zz = zz + bcat_ref[...]
    z0 = zz[0:A, :]
    z1 = zz[8:8 + A, :]
    base = zz[16:16 + A, :]
    m0 = jnp.max(z0, axis=0, keepdims=True)
    m1 = jnp.max(z1, axis=0, keepdims=True)
    mm = m0 + m1
    c0 = c0_ref[...]
    c1 = c1_ref[...]

    Bb = xb.shape[0]
    tag = lax.broadcasted_iota(jnp.int32, (A, Bb), 0)
    tag = (A - 1) - tag
    mask3 = jnp.int32(~7)
    acc = jnp.zeros((A, Bb), jnp.float32)
    wsum = jnp.zeros((1, Bb), jnp.float32)

    def pick(z, g):
        v = z + g
        vi = lax.bitcast_convert_type(v, jnp.int32)
        vk = lax.bitcast_convert_type((vi & mask3) | tag, jnp.float32)
        t = jnp.max(vk, axis=0, keepdims=True)
        return (vk == t).astype(jnp.float32)

    def usum(oh, z):
        return jnp.sum(oh * z, axis=0, keepdims=True)

    for i in range(S):
        oh0 = pick(z0, g_ref[0, i, :, :])
        oh1 = pick(z1, g_ref[0, S + i, :, :])
        w = jnp.exp(usum(oh0, z0) + usum(oh1, z1) - mm)
        l = base + jnp.dot(c0, oh0, preferred_element_type=jnp.float32) \
                 + jnp.dot(c1, oh1, preferred_element_type=jnp.float32)
        e = jnp.exp(l)
        s = jnp.sum(e, axis=0, keepdims=True)
        acc = acc + (w / s) * e
        wsum = wsum + w

    out_ref[...] = (acc / wsum).T


def _tc_call(x, wcat, bcat, c0, c1, g, Bb):
    B, D = x.shape
    A, S = _A, _S
    return pl.pallas_call(
        _tc_body,
        grid=(B // Bb,),
        in_specs=[
            pl.BlockSpec((Bb, D), lambda i: (i, 0)),
            pl.BlockSpec((24, D), lambda i: (0, 0)),
            pl.BlockSpec((24, 1), lambda i: (0, 0)),
            pl.BlockSpec((A, A), lambda i: (0, 0)),
            pl.BlockSpec((A, A), lambda i: (0, 0)),
            pl.BlockSpec((1, _OPP * S, A, Bb), lambda i: (i, 0, 0, 0)),
        ],
        out_specs=pl.BlockSpec((Bb, A), lambda i: (i, 0)),
        out_shape=jax.ShapeDtypeStruct((B, A), jnp.float32),
        compiler_params=pltpu.CompilerParams(
            dimension_semantics=("parallel",),
        ),
    )(x, wcat, bcat, c0, c1, g)


def _zz_body(xb_ref, wcat_ref, bcat_ref, out_ref):
    zz = lax.dot_general(wcat_ref[...], xb_ref[...], (((1,), (1,)), ((), ())),
                         preferred_element_type=jnp.float32)
    out_ref[...] = zz + bcat_ref[...]


def _zz_call(x, wcat, bcat):
    B, D = x.shape
    Bb = 2048
    return pl.pallas_call(
        _zz_body,
        grid=(B // Bb,),
        in_specs=[
            pl.BlockSpec((Bb, D), lambda i: (i, 0)),
            pl.BlockSpec((24, D), lambda i: (0, 0)),
            pl.BlockSpec((24, 1), lambda i: (0, 0)),
        ],
        out_specs=pl.BlockSpec((24, Bb), lambda i: (0, i)),
        out_shape=jax.ShapeDtypeStruct((24, B), jnp.float32),
        compiler_params=pltpu.CompilerParams(
            dimension_semantics=("parallel",),
        ),
    )(x, wcat, bcat)


_NC = 2
_NS = 16
_NW = _NC * _NS
_CH = 128


def _sc_body(zz_hbm, g_hbm, ptab_hbm, out_hbm, zz_v, g_v, ptab_v, out_v):
    A, S = _A, _S
    wid = lax.axis_index("s") * _NC + lax.axis_index("c")
    B = zz_hbm.shape[1]
    rpw = B // _NW
    base = wid * rpw

    pltpu.sync_copy(ptab_hbm, ptab_v)
    pltpu.sync_copy(zz_hbm.at[:, pl.ds(base, rpw)], zz_v)

    lane = lax.iota(jnp.int32, 16)

    for c in range(rpw // _CH):
        coff = c * _CH
        pltpu.sync_copy(g_hbm.at[:, pl.ds(base + coff, _CH)], g_v)

        def grp(gi, carry):
            goff = coff + gi * 16
            gg = gi * 16
            z0 = [zz_v[a, pl.ds(goff, 16)] for a in range(A)]
            z1 = [zz_v[8 + a, pl.ds(goff, 16)] for a in range(A)]
            bs = [zz_v[16 + o, pl.ds(goff, 16)] for o in range(A)]
            m0 = z0[0]
            m1 = z1[0]
            for a in range(1, A):
                m0 = jnp.maximum(m0, z0[a])
                m1 = jnp.maximum(m1, z1[a])
            mm = m0 + m1
            acc = [jnp.zeros((16,), jnp.float32) for _ in range(A)]
            wsum = jnp.zeros((16,), jnp.float32)

            def pick(zrows, irow):
                v = [zrows[a] + g_v[irow * A + a, pl.ds(gg, 16)]
                     for a in range(A)]
                m = v[0]
                for a in range(1, A):
                    m = jnp.maximum(m, v[a])
                aidx = jnp.full((16,), A - 1, jnp.int32)
                u = zrows[A - 1]
                for a in range(A - 2, -1, -1):
                    hit = v[a] == m
                    aidx = jnp.where(hit, jnp.int32(a), aidx)
                    u = jnp.where(hit, zrows[a], u)
                return aidx, u

            for i in range(S):
                a0, u0 = pick(z0, i)
                a1, u1 = pick(z1, S + i)
                w = jnp.exp(u0 + u1 - mm)
                pidx = a0 * A + a1
                e = []
                s = None
                for o in range(A):
                    po = plsc.load_gather(ptab_v, [pidx + (o * A * A)])
                    eo = jnp.exp(bs[o] + po)
                    e.append(eo)
                    s = eo if s is None else s + eo
                r = w / s
                for o in range(A):
                    acc[o] = acc[o] + r * e[o]
                wsum = wsum + w

            orow = (goff + lane) * A
            for o in range(A):
                plsc.store_scatter(out_v, [orow + o], acc[o] / wsum)
            return carry

        lax.fori_loop(0, _CH // 16, grp, 0)

    pltpu.sync_copy(out_v, out_hbm.at[pl.ds(base * A, rpw * A)])


def _sc_call(zz, g, ptab):
    B = zz.shape[1]
    rpw = B // _NW
    mesh = plsc.VectorSubcoreMesh(core_axis_name="c", subcore_axis_name="s")
    k = functools.partial(
        pl.kernel,
        out_type=jax.ShapeDtypeStruct((B * _A,), jnp.float32),
        mesh=mesh,
        scratch_types=[
            pltpu.VMEM((24, rpw), jnp.float32),
            pltpu.VMEM((_OPP * _S * _A, _CH), jnp.float32),
            pltpu.VMEM((_A * _A * _A,), jnp.float32),
            pltpu.VMEM((rpw * _A,), jnp.float32),
        ],
        compiler_params=pltpu.CompilerParams(needs_layout_passes=False),
    )(_sc_body)
    return k(zz, g, ptab)


def kernel(x, W_opp0, b_opp0, W_opp1, b_opp1, W, b):
    B, D = x.shape
    A = _A
    Bb = 2048
    Bsc = 4096
    Bt = B - Bsc

    zpadW = jnp.zeros((2, D), x.dtype)
    wcat = jnp.concatenate(
        [W_opp0, zpadW, W_opp1, zpadW, W[:, :D], zpadW], axis=0)
    zpadb = jnp.zeros((2,), x.dtype)
    bcat = jnp.concatenate(
        [b_opp0, zpadb, b_opp1, zpadb, b, zpadb], axis=0)[:, None]
    c0 = W[:, D:D + A]
    c1 = W[:, D + A:D + 2 * A]
    ptab = (c0[:, :, None] + c1[:, None, :]).reshape(A * A * A)

    g_tc = jnp.asarray(_gumbel_host_tc(B, 0, Bt, Bb))
    g_sc = jnp.asarray(_gumbel_host_sc(B, Bt, B))

    zz = _zz_call(x[Bt:], wcat, bcat)
    out_sc = _sc_call(zz, g_sc, ptab)
    out_tc = _tc_call(x[:Bt], wcat, bcat, c0, c1, g_tc, Bb)
    return jnp.concatenate([out_tc, out_sc.reshape(Bsc, A)], axis=0)


# speedup vs baseline: 1.9158x; 1.3975x over previous
import functools

import jax
import jax.numpy as jnp
import numpy as np
from jax import lax
from jax.experimental import pallas as pl
from jax.experimental.pallas import tpu as pltpu

_A = 6
_S = 20
_OPP = 2


def _rotl(x, r):
    return (x << np.uint32(r)) | (x >> np.uint32(32 - r))


def _threefry2x32(k1, k2, x0, x1):
    ks0 = np.uint32(k1)
    ks1 = np.uint32(k2)
    ks2 = np.uint32(ks0 ^ ks1 ^ np.uint32(0x1BD11BDA))
    ks = [ks0, ks1, ks2]
    rotations = [(13, 15, 26, 6), (17, 29, 16, 24)]
    x0 = x0 + ks0
    x1 = x1 + ks1
    for i in range(5):
        for r in rotations[i % 2]:
            x0 = x0 + x1
            x1 = _rotl(x1, r)
            x1 = x1 ^ x0
        x0 = x0 + ks[(i + 1) % 3]
        x1 = x1 + ks[(i + 2) % 3] + np.uint32(i + 1)
    return x0, x1


def _fold_in(key, data):
    o0, o1 = _threefry2x32(key[0], key[1],
                           np.atleast_1d(np.uint32(0)),
                           np.atleast_1d(np.uint32(data)))
    return (o0[0], o1[0])


def _gumbel_np(key, n):
    cnt = np.arange(n, dtype=np.uint64)
    hi = (cnt >> np.uint64(32)).astype(np.uint32)
    lo = (cnt & np.uint64(0xFFFFFFFF)).astype(np.uint32)
    o0, o1 = _threefry2x32(key[0], key[1], hi, lo)
    bits = o0 ^ o1
    f = ((bits >> np.uint32(9)) | np.uint32(0x3F800000)).view(np.float32)
    u = f - np.float32(1.0)
    tiny = np.float32(np.finfo(np.float32).tiny)
    u = np.maximum(tiny, u * (np.float32(1.0) - tiny) + tiny)
    with np.errstate(divide="ignore"):
        return -np.log(-np.log(u))


@functools.lru_cache(maxsize=2)
def _gumbel_host(B, Bb):
    root = (np.uint32(0), np.uint32(42))
    nb = B // Bb
    out = np.empty((nb, _OPP * _S, _A, Bb), np.float32)
    for j in range(_OPP):
        kj = _fold_in(root, j)
        for i in range(_S):
            ki = _fold_in(kj, i)
            g = _gumbel_np(ki, B * _A).reshape(nb, Bb, _A)
            out[:, j * _S + i] = g.transpose(0, 2, 1)
    return out


def _body(xb_ref, wcat_ref, bcat_ref, mfuse_ref, ones6_ref, g_ref, out_ref):
    A, S = _A, _S
    xb = xb_ref[...]
    zz = lax.dot_general(wcat_ref[...], xb, (((1,), (1,)), ((), ())),
                         preferred_element_type=jnp.float32)
    zz = zz + bcat_ref[...]
    z0 = zz[0:A, :]
    z1 = zz[8:8 + A, :]
    base = zz[16:16 + A, :]
    m0 = jnp.max(z0, axis=0, keepdims=True)
    m1 = jnp.max(z1, axis=0, keepdims=True)
    mm = m0 + m1
    mfuse = mfuse_ref[...]
    ones6 = ones6_ref[...]

    Bb = xb.shape[0]
    zero2 = jnp.zeros((2, Bb), jnp.float32)
    tag = lax.broadcasted_iota(jnp.int32, (A, Bb), 0)
    tag = (A - 1) - tag
    mask3 = jnp.int32(~7)
    acc = jnp.zeros((A, Bb), jnp.float32)
    wsum = jnp.zeros((1, Bb), jnp.float32)

    def pick(z, g):
        v = z + g
        vi = lax.bitcast_convert_type(v, jnp.int32)
        vk = lax.bitcast_convert_type((vi & mask3) | tag, jnp.float32)
        t = jnp.max(vk, axis=0, keepdims=True)
        oh = (vk == t).astype(jnp.float32)
        return oh

    c0 = mfuse_ref[0:A, 0:A]
    c1 = mfuse_ref[0:A, 16:16 + A]

    def usum(oh, z):
        return jnp.sum(oh * z, axis=0, keepdims=True)

    for i in range(S):
        oh0 = pick(z0, g_ref[0, i, :, :])
        oh1 = pick(z1, g_ref[0, S + i, :, :])
        w = jnp.exp(usum(oh0, z0) + usum(oh1, z1) - mm)
        l = base + jnp.dot(c0, oh0, preferred_element_type=jnp.float32) \
                 + jnp.dot(c1, oh1, preferred_element_type=jnp.float32)
        e = jnp.exp(l)
        s = jnp.sum(e, axis=0, keepdims=True)
        acc = acc + (w / s) * e
        wsum = wsum + w

    eye = (lax.broadcasted_iota(jnp.int32, (A, A), 0)
           == lax.broadcasted_iota(jnp.int32, (A, A), 1)).astype(jnp.float32)
    out_ref[...] = lax.dot_general(acc / wsum, eye, (((0,), (0,)), ((), ())),
                                   preferred_element_type=jnp.float32)


def kernel(x, W_opp0, b_opp0, W_opp1, b_opp1, W, b):
    B, D = x.shape
    A, S = _A, _S

    Bb = 2048
    nb = B // Bb
    g = jnp.asarray(_gumbel_host(B, Bb))

    zpadW = jnp.zeros((2, D), x.dtype)
    wcat = jnp.concatenate(
        [W_opp0, zpadW, W_opp1, zpadW, W[:, :D], zpadW], axis=0)
    zpadb = jnp.zeros((2,), x.dtype)
    bcat = jnp.concatenate(
        [b_opp0, zpadb, b_opp1, zpadb, b, zpadb], axis=0)[:, None]
    c0 = W[:, D:D + A]
    c1 = W[:, D + A:D + 2 * A]
    mfuse = jnp.zeros((8, 32), jnp.float32)
    mfuse = mfuse.at[0:A, 0:A].set(c0)
    mfuse = mfuse.at[0:A, 16:16 + A].set(c1)
    mfuse = mfuse.at[6, 8:8 + A].set(1.0)
    mfuse = mfuse.at[6, 24:24 + A].set(1.0)
    ones6 = jnp.ones((1, A), jnp.float32)

    out = pl.pallas_call(
        _body,
        grid=(nb,),
        in_specs=[
            pl.BlockSpec((Bb, D), lambda i: (i, 0)),
            pl.BlockSpec((24, D), lambda i: (0, 0)),
            pl.BlockSpec((24, 1), lambda i: (0, 0)),
            pl.BlockSpec((8, 32), lambda i: (0, 0)),
            pl.BlockSpec((1, A), lambda i: (0, 0)),
            pl.BlockSpec((1, _OPP * S, A, Bb), lambda i: (i, 0, 0, 0)),
        ],
        out_specs=pl.BlockSpec((Bb, A), lambda i: (i, 0)),
        out_shape=jax.ShapeDtypeStruct((B, A), jnp.float32),
        compiler_params=pltpu.CompilerParams(
            dimension_semantics=("parallel",),
        ),
    )(x, wcat, bcat, mfuse, ones6, g)

    return out


# speedup vs baseline: 2.7369x; 1.4286x over previous
import functools

import jax
import jax.numpy as jnp
import numpy as np
from jax import lax
from jax.experimental import pallas as pl
from jax.experimental.pallas import tpu as pltpu

_A = 6
_S = 20
_OPP = 2


def _rotl(x, r):
    return (x << np.uint32(r)) | (x >> np.uint32(32 - r))


def _threefry2x32(k1, k2, x0, x1):
    ks0 = np.uint32(k1)
    ks1 = np.uint32(k2)
    ks2 = np.uint32(ks0 ^ ks1 ^ np.uint32(0x1BD11BDA))
    ks = [ks0, ks1, ks2]
    rotations = [(13, 15, 26, 6), (17, 29, 16, 24)]
    x0 = x0 + ks0
    x1 = x1 + ks1
    for i in range(5):
        for r in rotations[i % 2]:
            x0 = x0 + x1
            x1 = _rotl(x1, r)
            x1 = x1 ^ x0
        x0 = x0 + ks[(i + 1) % 3]
        x1 = x1 + ks[(i + 2) % 3] + np.uint32(i + 1)
    return x0, x1


def _fold_in(key, data):
    o0, o1 = _threefry2x32(key[0], key[1],
                           np.atleast_1d(np.uint32(0)),
                           np.atleast_1d(np.uint32(data)))
    return (o0[0], o1[0])


def _gumbel_np(key, n):
    cnt = np.arange(n, dtype=np.uint64)
    hi = (cnt >> np.uint64(32)).astype(np.uint32)
    lo = (cnt & np.uint64(0xFFFFFFFF)).astype(np.uint32)
    o0, o1 = _threefry2x32(key[0], key[1], hi, lo)
    bits = o0 ^ o1
    f = ((bits >> np.uint32(9)) | np.uint32(0x3F800000)).view(np.float32)
    u = f - np.float32(1.0)
    tiny = np.float32(np.finfo(np.float32).tiny)
    u = np.maximum(tiny, u * (np.float32(1.0) - tiny) + tiny)
    with np.errstate(divide="ignore"):
        return -np.log(-np.log(u))


@functools.lru_cache(maxsize=2)
def _gumbel_host_b(B, Bb):
    root = (np.uint32(0), np.uint32(42))
    nb = B // Bb
    out = np.empty((nb, _OPP * _A, _S, Bb), np.float32)
    for j in range(_OPP):
        kj = _fold_in(root, j)
        for i in range(_S):
            ki = _fold_in(kj, i)
            g = _gumbel_np(ki, B * _A).reshape(nb, Bb, _A)
            for a in range(_A):
                out[:, j * _A + a, i, :] = g[:, :, a]
    return out


def _body_b(xb_ref, wcat_ref, bcat_ref, c0_ref, c1_ref, g_ref, out_ref):
    A, S = _A, _S
    xb = xb_ref[...]
    zz = lax.dot_general(wcat_ref[...], xb, (((1,), (1,)), ((), ())),
                         preferred_element_type=jnp.float32)
    zz = zz + bcat_ref[...]
    Bb = xb.shape[0]

    z0 = [zz[a:a + 1, :] for a in range(A)]
    z1 = [zz[8 + a:9 + a, :] for a in range(A)]
    bs = [zz[16 + o:17 + o, :] for o in range(A)]
    m0 = z0[0]
    m1 = z1[0]
    for a in range(1, A):
        m0 = jnp.maximum(m0, z0[a])
        m1 = jnp.maximum(m1, z1[a])
    mm = m0 + m1

    c0 = c0_ref[...]
    c1 = c1_ref[...]

    def pick(zrows, goff):
        v = [zrows[a] + g_ref[0, goff + a, :, :] for a in range(A)]
        m = v[0]
        for a in range(1, A):
            m = jnp.maximum(m, v[a])
        hit = [v[a] == m for a in range(A)]
        u = jnp.broadcast_to(zrows[A - 1], (S, Bb))
        for a in range(A - 2, -1, -1):
            u = jnp.where(hit[a], zrows[a], u)
        return hit, u

    hit0, u0 = pick(z0, 0)
    hit1, u1 = pick(z1, A)
    w = jnp.exp(u0 + u1 - mm)

    s = None
    e = []
    for o in range(A):
        d = jnp.broadcast_to(c0[o, A - 1] + bs[o], (S, Bb))
        for a in range(A - 2, -1, -1):
            d = jnp.where(hit0[a], c0[o, a] + bs[o], d)
        dd = jnp.broadcast_to(c1[o, A - 1], (S, Bb))
        for a in range(A - 2, -1, -1):
            dd = jnp.where(hit1[a], c1[o, a], dd)
        eo = jnp.exp(d + dd)
        e.append(eo)
        s = eo if s is None else s + eo
    r = w / s
    wsum = jnp.sum(w, axis=0, keepdims=True)
    outs = []
    for o in range(A):
        outs.append(jnp.sum(r * e[o], axis=0, keepdims=True) / wsum)
    out_ref[...] = jnp.concatenate(outs, axis=0).T


def kernel(x, W_opp0, b_opp0, W_opp1, b_opp1, W, b):
    B, D = x.shape
    A, S = _A, _S

    Bb = 2048
    nb = B // Bb
    g = jnp.asarray(_gumbel_host_b(B, Bb))

    zpadW = jnp.zeros((2, D), x.dtype)
    wcat = jnp.concatenate(
        [W_opp0, zpadW, W_opp1, zpadW, W[:, :D], zpadW], axis=0)
    zpadb = jnp.zeros((2,), x.dtype)
    bcat = jnp.concatenate(
        [b_opp0, zpadb, b_opp1, zpadb, b, zpadb], axis=0)[:, None]
    c0 = W[:, D:D + A]
    c1 = W[:, D + A:D + 2 * A]

    out = pl.pallas_call(
        _body_b,
        grid=(nb,),
        in_specs=[
            pl.BlockSpec((Bb, D), lambda i: (i, 0)),
            pl.BlockSpec((24, D), lambda i: (0, 0)),
            pl.BlockSpec((24, 1), lambda i: (0, 0)),
            pl.BlockSpec((A, A), lambda i: (0, 0)),
            pl.BlockSpec((A, A), lambda i: (0, 0)),
            pl.BlockSpec((1, _OPP * A, S, Bb), lambda i: (i, 0, 0, 0)),
        ],
        out_specs=pl.BlockSpec((Bb, A), lambda i: (i, 0)),
        out_shape=jax.ShapeDtypeStruct((B, A), jnp.float32),
        compiler_params=pltpu.CompilerParams(
            dimension_semantics=("parallel",),
        ),
    )(x, wcat, bcat, c0, c1, g)
    return out


# speedup vs baseline: 2.8777x; 1.0514x over previous
import functools

import jax
import jax.numpy as jnp
import numpy as np
from jax import lax
from jax.experimental import pallas as pl
from jax.experimental.pallas import tpu as pltpu

_A = 6
_S = 20
_OPP = 2


def _rotl(x, r):
    return (x << np.uint32(r)) | (x >> np.uint32(32 - r))


def _threefry2x32(k1, k2, x0, x1):
    ks0 = np.uint32(k1)
    ks1 = np.uint32(k2)
    ks2 = np.uint32(ks0 ^ ks1 ^ np.uint32(0x1BD11BDA))
    ks = [ks0, ks1, ks2]
    rotations = [(13, 15, 26, 6), (17, 29, 16, 24)]
    x0 = x0 + ks0
    x1 = x1 + ks1
    for i in range(5):
        for r in rotations[i % 2]:
            x0 = x0 + x1
            x1 = _rotl(x1, r)
            x1 = x1 ^ x0
        x0 = x0 + ks[(i + 1) % 3]
        x1 = x1 + ks[(i + 2) % 3] + np.uint32(i + 1)
    return x0, x1


def _fold_in(key, data):
    o0, o1 = _threefry2x32(key[0], key[1],
                           np.atleast_1d(np.uint32(0)),
                           np.atleast_1d(np.uint32(data)))
    return (o0[0], o1[0])


def _gumbel_np(key, n):
    cnt = np.arange(n, dtype=np.uint64)
    hi = (cnt >> np.uint64(32)).astype(np.uint32)
    lo = (cnt & np.uint64(0xFFFFFFFF)).astype(np.uint32)
    o0, o1 = _threefry2x32(key[0], key[1], hi, lo)
    bits = o0 ^ o1
    f = ((bits >> np.uint32(9)) | np.uint32(0x3F800000)).view(np.float32)
    u = f - np.float32(1.0)
    tiny = np.float32(np.finfo(np.float32).tiny)
    u = np.maximum(tiny, u * (np.float32(1.0) - tiny) + tiny)
    with np.errstate(divide="ignore"):
        return -np.log(-np.log(u))


@functools.lru_cache(maxsize=2)
def _gumbel_host_b(B, Bb):
    root = (np.uint32(0), np.uint32(42))
    nb = B // Bb
    out = np.empty((nb, _OPP * _A, _S, Bb), np.float32)
    for j in range(_OPP):
        kj = _fold_in(root, j)
        for i in range(_S):
            ki = _fold_in(kj, i)
            g = _gumbel_np(ki, B * _A).reshape(nb, Bb, _A)
            for a in range(_A):
                out[:, j * _A + a, i, :] = g[:, :, a]
    return out


def _body_b(xb_ref, wcat_ref, bcat_ref, c0_ref, c1_ref, g_ref, out_ref):
    A, S = _A, _S
    xb = xb_ref[...]
    zz = lax.dot_general(wcat_ref[...], xb, (((1,), (1,)), ((), ())),
                         preferred_element_type=jnp.float32)
    zz = zz + bcat_ref[...]
    Bb = xb.shape[0]

    z0 = [zz[a:a + 1, :] for a in range(A)]
    z1 = [zz[8 + a:9 + a, :] for a in range(A)]
    bs = [zz[16 + o:17 + o, :] for o in range(A)]
    m0 = z0[0]
    m1 = z1[0]
    for a in range(1, A):
        m0 = jnp.maximum(m0, z0[a])
        m1 = jnp.maximum(m1, z1[a])
    mm = m0 + m1

    c0 = c0_ref[...]
    c1 = c1_ref[...]

    def pick(zrows, goff):
        v = [zrows[a] + g_ref[0, goff + a, :, :] for a in range(A)]
        m = v[0]
        for a in range(1, A):
            m = jnp.maximum(m, v[a])
        hit = [v[a] == m for a in range(A)]
        u = jnp.broadcast_to(zrows[A - 1], (S, Bb))
        for a in range(A - 2, -1, -1):
            u = jnp.where(hit[a], zrows[a], u)
        return hit, u

    hit0, u0 = pick(z0, 0)
    hit1, u1 = pick(z1, A)
    w = jnp.exp(u0 + u1 - mm)

    s = None
    e = []
    for o in range(A):
        ebs = jnp.exp(bs[o])
        d = jnp.broadcast_to(c0[o, A - 1] * ebs, (S, Bb))
        for a in range(A - 2, -1, -1):
            d = jnp.where(hit0[a], c0[o, a] * ebs, d)
        dd = jnp.broadcast_to(c1[o, A - 1], (S, Bb))
        for a in range(A - 2, -1, -1):
            dd = jnp.where(hit1[a], c1[o, a], dd)
        eo = d * dd
        e.append(eo)
        s = eo if s is None else s + eo
    r = w / s
    wsum = jnp.sum(w, axis=0, keepdims=True)
    outs = []
    for o in range(A):
        outs.append(jnp.sum(r * e[o], axis=0, keepdims=True) / wsum)
    out_ref[...] = jnp.concatenate(outs, axis=0).T


def kernel(x, W_opp0, b_opp0, W_opp1, b_opp1, W, b):
    B, D = x.shape
    A, S = _A, _S

    Bb = 4096
    nb = B // Bb
    g = jnp.asarray(_gumbel_host_b(B, Bb))

    zpadW = jnp.zeros((2, D), x.dtype)
    wcat = jnp.concatenate(
        [W_opp0, zpadW, W_opp1, zpadW, W[:, :D], zpadW], axis=0)
    zpadb = jnp.zeros((2,), x.dtype)
    bcat = jnp.concatenate(
        [b_opp0, zpadb, b_opp1, zpadb, b, zpadb], axis=0)[:, None]
    c0 = jnp.exp(W[:, D:D + A])
    c1 = jnp.exp(W[:, D + A:D + 2 * A])

    out = pl.pallas_call(
        _body_b,
        grid=(nb,),
        in_specs=[
            pl.BlockSpec((Bb, D), lambda i: (i, 0)),
            pl.BlockSpec((24, D), lambda i: (0, 0)),
            pl.BlockSpec((24, 1), lambda i: (0, 0)),
            pl.BlockSpec((A, A), lambda i: (0, 0)),
            pl.BlockSpec((A, A), lambda i: (0, 0)),
            pl.BlockSpec((1, _OPP * A, S, Bb), lambda i: (i, 0, 0, 0)),
        ],
        out_specs=pl.BlockSpec((Bb, A), lambda i: (i, 0)),
        out_shape=jax.ShapeDtypeStruct((B, A), jnp.float32),
        compiler_params=pltpu.CompilerParams(
            dimension_semantics=("parallel",),
        ),
    )(x, wcat, bcat, c0, c1, g)
    return out
